# Initial kernel scaffold; baseline (speedup 1.0000x reference)
import jax
import jax.numpy as jnp
from jax.experimental import pallas as pl


def kernel(x, embeds):
    raise NotImplementedError("write your pallas kernel here")


# speedup vs baseline: 1.0934x; 1.0934x over previous
import functools

import jax
import jax.numpy as jnp
from jax import lax
from jax.experimental import pallas as pl
from jax.experimental.pallas import tpu as pltpu
from jax.experimental.pallas import tpu_sc as plsc

_N = 18432
_D = 256
_K = 8192
_TN = 256
_T = _N // _TN

_f32 = jnp.float32
_DOT_DIMS = (((1,), (0,)), ((), ()))

_SLABS_2048 = [(0, 2048), (2048, 4096), (4096, 6144), (6144, 8192)]
_SLABS_2816 = [(0, 2816), (2816, 5632), (5632, 8192)]


def _prep_codebook(e_ref, ehi_ref, e2_ref):
    e = e_ref[...]
    ehi_ref[...] = e.astype(jnp.bfloat16)
    e2_ref[...] = jnp.sum(e * e, axis=0, keepdims=True)


def _argmin_dist(rt, a, ehi, e2, slabs):
    rb = rt.astype(jnp.bfloat16)
    s = lax.dot_general(rb, ehi, _DOT_DIMS, preferred_element_type=_f32)
    neg = -((a - 2.0 * s) + e2)
    acc = jnp.full((rt.shape[0], 1), -jnp.inf, _f32)
    idx = jnp.zeros((rt.shape[0], 1), jnp.int32)
    for lo, hi in slabs:
        blk = neg[:, lo:hi]
        m = jnp.max(blk, axis=1, keepdims=True)
        io = lax.broadcasted_iota(jnp.int32, blk.shape, 1)
        i = jnp.min(jnp.where(blk == m, io, _K), axis=1, keepdims=True) + lo
        take = m > acc
        acc = jnp.where(take, m.astype(jnp.bfloat16).astype(_f32), acc)
        idx = jnp.where(take, i, idx)
    return idx


def _dist_body(slabs, r_ref, a_ref, e_ref, ind_ref, ehi, e2):
    @pl.when(pl.program_id(0) == 0)
    def _():
        _prep_codebook(e_ref, ehi, e2)

    ind_ref[...] = _argmin_dist(r_ref[...], a_ref[...], ehi[...], e2[...],
                                slabs)


def _quant_body(q0_ref, q1_ref, q2_ref, q3_ref, out_ref):
    out_ref[...] = ((q0_ref[...] + q1_ref[...]) + q2_ref[...]) + q3_ref[...]


_SCRATCH = [
    pltpu.VMEM((_D, _K), jnp.bfloat16),
    pltpu.VMEM((1, _K), _f32),
]

_ROW_SPEC = pl.BlockSpec((_TN, _D), lambda t: (t, 0))
_A_SPEC = pl.BlockSpec((_TN, 1), lambda t: (t, 0))
_E_SPEC = pl.BlockSpec((_D, _K), lambda t: (0, 0))
_IND_SPEC = pl.BlockSpec((_TN, 1), lambda t: (t, 0))


def _dist(r, a, e, slabs):
    return pl.pallas_call(
        functools.partial(_dist_body, slabs),
        grid=(_T,),
        in_specs=[_ROW_SPEC, _A_SPEC, _E_SPEC],
        out_specs=_IND_SPEC,
        out_shape=jax.ShapeDtypeStruct((_N, 1), jnp.int32),
        scratch_shapes=_SCRATCH,
    )(r, a, e)


def _quant_sum(q0, q1, q2, q3):
    return pl.pallas_call(
        _quant_body,
        grid=(_T,),
        in_specs=[_ROW_SPEC, _ROW_SPEC, _ROW_SPEC, _ROW_SPEC],
        out_specs=_ROW_SPEC,
        out_shape=jax.ShapeDtypeStruct((_N, _D), _f32),
    )(q0, q1, q2, q3)


def _sc_geometry():
    try:
        info = plsc.get_sparse_core_info()
        nc, ns = info.num_cores, info.num_subcores
    except Exception:
        nc, ns = 2, 16
    return nc, ns


def _make_gather():
    nc, ns = _sc_geometry()
    nw = nc * ns
    bpw = _N // nw
    chunks = 2
    crows = bpw // chunks
    mesh = plsc.VectorSubcoreMesh(core_axis_name="c", subcore_axis_name="s")

    @functools.partial(
        pl.kernel, mesh=mesh,
        out_type=jax.ShapeDtypeStruct((_N, _D), _f32),
        scratch_types=[pltpu.VMEM((crows,), jnp.int32),
                       pltpu.VMEM((crows, _D), _f32),
                       pltpu.SemaphoreType.DMA],
    )
    def gk(table_hbm, idx_hbm, out_hbm, idx_v, rows_v, sem):
        wid = lax.axis_index("s") * nc + lax.axis_index("c")
        base = wid * bpw
        for c in range(chunks):
            pltpu.sync_copy(idx_hbm.at[wid, c], idx_v)
            pltpu.async_copy(table_hbm.at[idx_v], rows_v, sem).wait()
            pltpu.sync_copy(rows_v, out_hbm.at[pl.ds(base + c * crows, crows)])

    return gk, nw, chunks, crows


def kernel(x, embeds):
    e_t = jnp.swapaxes(embeds, 1, 2)
    gather, nw, chunks, crows = _make_gather()

    def idx3(ind):
        return ind.reshape(nw, chunks, crows)

    slab_sets = [_SLABS_2048, _SLABS_2048, _SLABS_2048, _SLABS_2816]
    residual = x
    total_loss = jnp.asarray(0.0, dtype=_f32)
    inds = []
    qs = []
    for lvl in range(4):
        a = jnp.sum(residual ** 2, axis=1, keepdims=True)
        ind = _dist(residual, a, embeds[lvl], slab_sets[lvl])
        q = gather(e_t[lvl], idx3(ind))
        total_loss = total_loss + jnp.mean((q - residual) ** 2)
        residual = residual - q
        inds.append(ind[:, 0])
        qs.append(q)

    quant = _quant_sum(*qs)
    embed_inds = jnp.stack(inds, axis=0)
    return quant, total_loss, embed_inds


# speedup vs baseline: 1.1742x; 1.0739x over previous
import functools

import jax
import jax.numpy as jnp
from jax import lax
from jax.experimental import pallas as pl
from jax.experimental.pallas import tpu as pltpu
from jax.experimental.pallas import tpu_sc as plsc

_N = 18432
_D = 256
_K = 8192
_TN = 512
_T = _N // _TN

_f32 = jnp.float32
_DOT_DIMS = (((1,), (0,)), ((), ()))

_SLABS_2048 = [(0, 2048), (2048, 4096), (4096, 6144), (6144, 8192)]
_SLABS_2816 = [(0, 2816), (2816, 5632), (5632, 8192)]


def _prep_codebook(e_ref, ehi_ref, e2_ref):
    e = e_ref[...]
    ehi_ref[...] = e.astype(jnp.bfloat16)
    e2_ref[...] = jnp.sum(e * e, axis=0, keepdims=True)


def _argmin_dist(rt, a, ehi, e2, slabs):
    rb = rt.astype(jnp.bfloat16)
    s = lax.dot_general(rb, ehi, _DOT_DIMS, preferred_element_type=_f32)
    neg = -((a - 2.0 * s) + e2)
    acc = jnp.full((rt.shape[0], 1), -jnp.inf, _f32)
    idx = jnp.zeros((rt.shape[0], 1), jnp.int32)
    for lo, hi in slabs:
        blk = neg[:, lo:hi]
        m = jnp.max(blk, axis=1, keepdims=True)
        io = lax.broadcasted_iota(jnp.int32, blk.shape, 1)
        i = jnp.min(jnp.where(blk == m, io, _K), axis=1, keepdims=True) + lo
        take = m > acc
        acc = jnp.where(take, m.astype(jnp.bfloat16).astype(_f32), acc)
        idx = jnp.where(take, i, idx)
    return idx


def _dist_body(slabs, r_ref, a_ref, e_ref, ind_ref, ehi, e2):
    @pl.when(pl.program_id(0) == 0)
    def _():
        _prep_codebook(e_ref, ehi, e2)

    ind_ref[...] = _argmin_dist(r_ref[...], a_ref[...], ehi[...], e2[...],
                                slabs)


def _quant_body(q0_ref, q1_ref, q2_ref, q3_ref, out_ref):
    out_ref[...] = ((q0_ref[...] + q1_ref[...]) + q2_ref[...]) + q3_ref[...]


_SCRATCH = [
    pltpu.VMEM((_D, _K), jnp.bfloat16),
    pltpu.VMEM((1, _K), _f32),
]

_ROW_SPEC = pl.BlockSpec((_TN, _D), lambda t: (t, 0))
_A_SPEC = pl.BlockSpec((_TN, 1), lambda t: (t, 0))
_E_SPEC = pl.BlockSpec((_D, _K), lambda t: (0, 0))
_IND_SPEC = pl.BlockSpec((_TN, 1), lambda t: (t, 0))


def _dist(r, a, e, slabs):
    return pl.pallas_call(
        functools.partial(_dist_body, slabs),
        grid=(_T,),
        in_specs=[_ROW_SPEC, _A_SPEC, _E_SPEC],
        out_specs=_IND_SPEC,
        out_shape=jax.ShapeDtypeStruct((_N, 1), jnp.int32),
        scratch_shapes=_SCRATCH,
    )(r, a, e)


def _quant_sum(q0, q1, q2, q3):
    return pl.pallas_call(
        _quant_body,
        grid=(_T,),
        in_specs=[_ROW_SPEC, _ROW_SPEC, _ROW_SPEC, _ROW_SPEC],
        out_specs=_ROW_SPEC,
        out_shape=jax.ShapeDtypeStruct((_N, _D), _f32),
    )(q0, q1, q2, q3)


def _sc_geometry():
    try:
        info = plsc.get_sparse_core_info()
        nc, ns = info.num_cores, info.num_subcores
    except Exception:
        nc, ns = 2, 16
    return nc, ns


def _make_gather():
    nc, ns = _sc_geometry()
    nw = nc * ns
    bpw = _N // nw
    chunks = 2
    crows = bpw // chunks
    mesh = plsc.VectorSubcoreMesh(core_axis_name="c", subcore_axis_name="s")

    @functools.partial(
        pl.kernel, mesh=mesh,
        out_type=jax.ShapeDtypeStruct((_N, _D), _f32),
        scratch_types=[pltpu.VMEM((crows,), jnp.int32),
                       pltpu.VMEM((crows, _D), _f32),
                       pltpu.SemaphoreType.DMA],
    )
    def gk(table_hbm, idx_hbm, out_hbm, idx_v, rows_v, sem):
        wid = lax.axis_index("s") * nc + lax.axis_index("c")
        base = wid * bpw
        for c in range(chunks):
            pltpu.sync_copy(idx_hbm.at[wid, c], idx_v)
            pltpu.async_copy(table_hbm.at[idx_v], rows_v, sem).wait()
            pltpu.sync_copy(rows_v, out_hbm.at[pl.ds(base + c * crows, crows)])

    return gk, nw, chunks, crows


def kernel(x, embeds):
    e_t = jnp.swapaxes(embeds, 1, 2)
    gather, nw, chunks, crows = _make_gather()

    def idx3(ind):
        return ind.reshape(nw, chunks, crows)

    slab_sets = [_SLABS_2048, _SLABS_2048, _SLABS_2048, _SLABS_2816]
    residual = x
    total_loss = jnp.asarray(0.0, dtype=_f32)
    inds = []
    qs = []
    for lvl in range(4):
        a = jnp.sum(residual ** 2, axis=1, keepdims=True)
        ind = _dist(residual, a, embeds[lvl], slab_sets[lvl])
        q = gather(e_t[lvl], idx3(ind))
        total_loss = total_loss + jnp.mean((q - residual) ** 2)
        residual = residual - q
        inds.append(ind[:, 0])
        qs.append(q)

    quant = _quant_sum(*qs)
    embed_inds = jnp.stack(inds, axis=0)
    return quant, total_loss, embed_inds
